# Initial kernel scaffold; baseline (speedup 1.0000x reference)
import jax
import jax.numpy as jnp
from jax.experimental import pallas as pl


def kernel(a, b, samples_regions, da, db):
    raise NotImplementedError("write your pallas kernel here")


# speedup vs baseline: 53.3841x; 53.3841x over previous
import functools

import jax
import jax.numpy as jnp
from jax import lax
from jax.experimental import pallas as pl
from jax.experimental.pallas import tpu as pltpu
from jax.experimental.pallas import tpu_sc as plsc

E, R, S = 100, 100000, 16384
NC, NS, L = 2, 16, 16
NW = NC * NS
VCHUNK = 8192


def _body(a_hbm, b_hbm, srt_hbm, da_hbm, db_hbm, outa_hbm, outb_hbm,
          row_v, idx_v, val_v):
    wid = lax.axis_index("s") * NC + lax.axis_index("c")
    for k in range((E + NW - 1) // NW):
        e = wid + k * NW

        @pl.when(e < E)
        def _process():
            pltpu.sync_copy(srt_hbm.at[e], idx_v)
            for src, dst, vals in ((a_hbm, outa_hbm, da_hbm),
                                   (b_hbm, outb_hbm, db_hbm)):
                pltpu.sync_copy(src.at[e], row_v)
                for c in range(S // VCHUNK):
                    pltpu.sync_copy(vals.at[pl.ds(c * VCHUNK, VCHUNK)], val_v)

                    def _inner(i, carry, c=c):
                        idx = idx_v[pl.ds(c * VCHUNK + i * L, L)]
                        v = val_v[pl.ds(i * L, L)]
                        plsc.addupdate_scatter(row_v, [idx], v)
                        return carry

                    lax.fori_loop(0, VCHUNK // L, _inner, 0)
                pltpu.sync_copy(row_v, dst.at[e])


_scatter_update = pl.kernel(
    _body,
    out_type=[jax.ShapeDtypeStruct((E, R), jnp.float32),
              jax.ShapeDtypeStruct((E, R), jnp.float32)],
    mesh=plsc.VectorSubcoreMesh(core_axis_name="c", subcore_axis_name="s",
                                num_cores=NC, num_subcores=NS),
    scratch_types=[pltpu.VMEM((R,), jnp.float32),
                   pltpu.VMEM((S,), jnp.int32),
                   pltpu.VMEM((VCHUNK,), jnp.float32)],
    compiler_params=pltpu.CompilerParams(needs_layout_passes=False),
)


@jax.jit
def kernel(a, b, samples_regions, da, db):
    srt = samples_regions.T
    return tuple(_scatter_update(a, b, srt, da, db))
